# Initial kernel scaffold; baseline (speedup 1.0000x reference)
import jax
import jax.numpy as jnp
from jax.experimental import pallas as pl


def kernel(inputs, embeddings):
    raise NotImplementedError("write your pallas kernel here")


# speedup vs baseline: 1.9246x; 1.9246x over previous
import functools

import jax
import jax.numpy as jnp
from jax import lax
from jax.experimental import pallas as pl

_N_EMB = 1024
_DIM = 64
_TILE = 2048


def _vq_body(x_ref, emb_ref, q_ref, loss_ref):
    x = x_ref[...]
    emb = emb_ref[...]
    x2 = jnp.sum(x * x, axis=1, keepdims=True)
    e2 = jnp.sum(emb * emb, axis=1)
    dots = lax.dot_general(x, emb, (((1,), (1,)), ((), ())))
    dist = (x2 + e2[None, :]) - 2.0 * dots
    mins = jnp.min(dist, axis=1, keepdims=True)
    col = lax.broadcasted_iota(jnp.int32, dist.shape, 1)
    idx = jnp.min(jnp.where(dist == mins, col, _N_EMB), axis=1)
    onehot = (col == idx[:, None]).astype(jnp.float32)
    q = lax.dot_general(onehot, emb, (((1,), (0,)), ((), ())))
    q_ref[...] = q
    d = q - x
    partial = jnp.sum(d * d).reshape(1, 1)

    @pl.when(pl.program_id(0) == 0)
    def _init():
        loss_ref[...] = partial

    @pl.when(pl.program_id(0) != 0)
    def _acc():
        loss_ref[...] += partial


@functools.partial(jax.jit, static_argnames=("interpret",))
def _vq_call(flat, embeddings, interpret=False):
    n = flat.shape[0]
    grid = n // _TILE
    q, loss = pl.pallas_call(
        _vq_body,
        grid=(grid,),
        in_specs=[
            pl.BlockSpec((_TILE, _DIM), lambda i: (i, 0)),
            pl.BlockSpec((_N_EMB, _DIM), lambda i: (0, 0)),
        ],
        out_specs=[
            pl.BlockSpec((_TILE, _DIM), lambda i: (i, 0)),
            pl.BlockSpec((1, 1), lambda i: (0, 0)),
        ],
        out_shape=[
            jax.ShapeDtypeStruct((n, _DIM), jnp.float32),
            jax.ShapeDtypeStruct((1, 1), jnp.float32),
        ],
        interpret=interpret,
    )(flat, embeddings)
    return q, loss


def kernel(inputs, embeddings, interpret=False):
    flat = inputs.reshape(-1, _DIM)
    q, loss_sum = _vq_call(flat, embeddings, interpret=interpret)
    n_elem = flat.shape[0] * _DIM
    vq_loss = loss_sum[0, 0] * (2.0 / n_elem)
    quantized_out = q.reshape(inputs.shape)
    return (quantized_out, vq_loss)
